# baseline (speedup 1.0000x reference)
import functools

import jax
import jax.numpy as jnp
from jax import lax
from jax.experimental import pallas as pl
from jax.experimental.pallas import tpu as pltpu
from jax.experimental.pallas import tpu_sc as plsc

N_NODES = 10000
D_FEAT = 128
N_EDGES = 320000

NC = 2
NS = 16
NW = NC * NS

EDGES_PER_W = N_EDGES // NW
CHUNK = 80
N_CHUNKS = EDGES_PER_W // CHUNK
GROUP = 25

N_PAD = 10240
ROWS_PER_TILE = N_PAD // NS
ZROWS = 80
HROWS = N_PAD // D_FEAT

_LANES = 16


def _sc_body(x_hbm, src_hbm, dst_hbm, eye_hbm, psum_hbm, pcnt_hbm,
             sum_sh, cnt_sh, eye_sh, src_g, dst_g,
             src0, dst0, hi0, lo0, rows0, oh0, semx0, seme0, semss0, semsc0,
             src1, dst1, hi1, lo1, rows1, oh1, semx1, seme1, semss1, semsc1):
    c = lax.axis_index("c")
    s = lax.axis_index("s")
    w = s * NC + c

    zeros16 = jnp.zeros((_LANES,), jnp.float32)

    def _zrow(i, carry):
        for blk in range(D_FEAT // _LANES):
            rows0[i, pl.ds(blk * _LANES, _LANES)] = zeros16
        return carry

    lax.fori_loop(0, ZROWS, _zrow, 0)

    row_base = s * ROWS_PER_TILE
    for k in range(ROWS_PER_TILE // ZROWS):
        pltpu.sync_copy(rows0, sum_sh.at[pl.ds(row_base + k * ZROWS, ZROWS)])

    @pl.when(s < HROWS // 8)
    def _zero_cnt():
        pltpu.sync_copy(rows0.at[pl.ds(0, 8)], cnt_sh.at[pl.ds(s * 8, 8)])

    @pl.when(s == NS - 1)
    def _stage_eye():
        pltpu.sync_copy(eye_hbm, eye_sh)

    plsc.subcore_barrier()

    edge_base = w * EDGES_PER_W
    bufs = (
        (src0, dst0, hi0, lo0, rows0, oh0, semx0, seme0, semss0, semsc0),
        (src1, dst1, hi1, lo1, rows1, oh1, semx1, seme1, semss1, semsc1),
    )

    def _wait_scatters(buf):
        src_v, dst_v, hi_v, lo_v, rows_v, oh_v, semx, seme, semss, semsc = buf
        pltpu.make_async_copy(rows_v, sum_sh.at[dst_v], semss).wait()
        pltpu.make_async_copy(oh_v, cnt_sh.at[hi_v], semsc).wait()

    def _issue(i, buf, first):
        src_v, dst_v, hi_v, lo_v, rows_v, oh_v, semx, seme, semss, semsc = buf
        if not first:
            @pl.when(i >= 2)
            def _():
                _wait_scatters(buf)

        @pl.when(i % GROUP == 0)
        def _load_group():
            goff = edge_base + (i // GROUP) * (GROUP * CHUNK)
            pltpu.sync_copy(src_hbm.at[pl.ds(goff, GROUP * CHUNK)], src_g)
            pltpu.sync_copy(dst_hbm.at[pl.ds(goff, GROUP * CHUNK)], dst_g)

        pos = (i % GROUP) * CHUNK
        for j in range(CHUNK // _LANES):
            src16 = src_g[pl.ds(pos + j * _LANES, _LANES)]
            dst16 = dst_g[pl.ds(pos + j * _LANES, _LANES)]
            src_v[pl.ds(j * _LANES, _LANES)] = src16
            dst_v[pl.ds(j * _LANES, _LANES)] = dst16
            hi_v[pl.ds(j * _LANES, _LANES)] = dst16 >> 7
            lo_v[pl.ds(j * _LANES, _LANES)] = dst16 & 127
        pltpu.async_copy(x_hbm.at[src_v], rows_v, semx)
        pltpu.async_copy(eye_sh.at[lo_v], oh_v, seme)

    def _drain(buf):
        src_v, dst_v, hi_v, lo_v, rows_v, oh_v, semx, seme, semss, semsc = buf
        pltpu.make_async_copy(x_hbm.at[src_v], rows_v, semx).wait()
        pltpu.async_copy(rows_v, sum_sh.at[dst_v], semss, add=True)
        pltpu.make_async_copy(eye_sh.at[lo_v], oh_v, seme).wait()
        pltpu.async_copy(oh_v, cnt_sh.at[hi_v], semsc, add=True)

    _issue(0, bufs[0], True)

    def _pair(g, carry):
        i0 = 2 * g
        _issue(i0 + 1, bufs[1], False)
        _drain(bufs[0])
        _issue(i0 + 2, bufs[0], False)
        _drain(bufs[1])
        return carry

    lax.fori_loop(0, (N_CHUNKS - 1) // 2, _pair, 0)
    _drain(bufs[0])
    _wait_scatters(bufs[1])
    _wait_scatters(bufs[0])
    plsc.subcore_barrier()

    pltpu.sync_copy(sum_sh.at[pl.ds(row_base, ROWS_PER_TILE)],
                    psum_hbm.at[c, pl.ds(row_base, ROWS_PER_TILE)])

    @pl.when(s < HROWS // 8)
    def _dump_cnt():
        pltpu.sync_copy(cnt_sh.at[pl.ds(s * 8, 8)],
                        pcnt_hbm.at[c, pl.ds(s * 8, 8)])


_sc_call = functools.partial(
    pl.kernel,
    mesh=plsc.VectorSubcoreMesh(core_axis_name="c", subcore_axis_name="s"),
    out_type=[
        jax.ShapeDtypeStruct((NC, N_PAD, D_FEAT), jnp.float32),
        jax.ShapeDtypeStruct((NC, HROWS, D_FEAT), jnp.float32),
    ],
    scratch_types=[
        pltpu.VMEM_SHARED((N_PAD, D_FEAT), jnp.float32),
        pltpu.VMEM_SHARED((HROWS, D_FEAT), jnp.float32),
        pltpu.VMEM_SHARED((D_FEAT, D_FEAT), jnp.float32),
        pltpu.VMEM((GROUP * CHUNK,), jnp.int32),
        pltpu.VMEM((GROUP * CHUNK,), jnp.int32),
    ] + 2 * [
        pltpu.VMEM((CHUNK,), jnp.int32),
        pltpu.VMEM((CHUNK,), jnp.int32),
        pltpu.VMEM((CHUNK,), jnp.int32),
        pltpu.VMEM((CHUNK,), jnp.int32),
        pltpu.VMEM((CHUNK, D_FEAT), jnp.float32),
        pltpu.VMEM((CHUNK, D_FEAT), jnp.float32),
        pltpu.SemaphoreType.DMA,
        pltpu.SemaphoreType.DMA,
        pltpu.SemaphoreType.DMA,
        pltpu.SemaphoreType.DMA,
    ],
)(_sc_body)


_ROWS_BLK = 400


def _divide_body(p0_ref, p1_ref, c0_ref, c1_ref, out_ref):
    cnt = c0_ref[...] + c1_ref[...]
    out_ref[...] = (p0_ref[...] + p1_ref[...]) / jnp.maximum(cnt, 1.0)


def kernel(X, edge_index):
    src = edge_index[0]
    dst = edge_index[1]
    eye = jnp.eye(D_FEAT, dtype=jnp.float32)
    psum, pcnt = _sc_call(X, src, dst, eye)
    c0 = pcnt[0].reshape(N_PAD, 1)
    c1 = pcnt[1].reshape(N_PAD, 1)
    out = pl.pallas_call(
        _divide_body,
        grid=(N_NODES // _ROWS_BLK,),
        in_specs=[
            pl.BlockSpec((_ROWS_BLK, D_FEAT), lambda i: (i, 0)),
            pl.BlockSpec((_ROWS_BLK, D_FEAT), lambda i: (i, 0)),
            pl.BlockSpec((_ROWS_BLK, 1), lambda i: (i, 0)),
            pl.BlockSpec((_ROWS_BLK, 1), lambda i: (i, 0)),
        ],
        out_specs=pl.BlockSpec((_ROWS_BLK, D_FEAT), lambda i: (i, 0)),
        out_shape=jax.ShapeDtypeStruct((N_NODES, D_FEAT), jnp.float32),
    )(psum[0], psum[1], c0, c1)
    return out


# speedup vs baseline: 1.0218x; 1.0218x over previous
import functools

import jax
import jax.numpy as jnp
from jax import lax
from jax.experimental import pallas as pl
from jax.experimental.pallas import tpu as pltpu
from jax.experimental.pallas import tpu_sc as plsc

N_NODES = 10000
D_FEAT = 128
N_EDGES = 320000

NC = 2
NS = 16
NW = NC * NS

EDGES_PER_W = N_EDGES // NW
CHUNK = 80
N_CHUNKS = EDGES_PER_W // CHUNK

N_PAD = 10240
ROWS_PER_TILE = N_PAD // NS
ZROWS = 32
HROWS = N_PAD // D_FEAT

_LANES = 16


def _sc_body(x_hbm, src_hbm, dst_hbm, eye_hbm, psum_hbm, pcnt_hbm,
             sum_sh, cnt_sh, eye_sh, zbuf,
             src0, dst0, hi0, lo0, rows0, oh0, semx0, seme0, semss0, semsc0,
             src1, dst1, hi1, lo1, rows1, oh1, semx1, seme1, semss1, semsc1):
    c = lax.axis_index("c")
    s = lax.axis_index("s")
    w = s * NC + c

    zeros16 = jnp.zeros((_LANES,), jnp.float32)

    def _zrow(i, carry):
        for blk in range(D_FEAT // _LANES):
            zbuf[i, pl.ds(blk * _LANES, _LANES)] = zeros16
        return carry

    lax.fori_loop(0, ZROWS, _zrow, 0)

    row_base = s * ROWS_PER_TILE
    for k in range(ROWS_PER_TILE // ZROWS):
        pltpu.sync_copy(zbuf, sum_sh.at[pl.ds(row_base + k * ZROWS, ZROWS)])

    @pl.when(s < HROWS // 8)
    def _zero_cnt():
        pltpu.sync_copy(zbuf.at[pl.ds(0, 8)], cnt_sh.at[pl.ds(s * 8, 8)])

    @pl.when(s == NS - 1)
    def _stage_eye():
        pltpu.sync_copy(eye_hbm, eye_sh)

    plsc.subcore_barrier()

    edge_base = w * EDGES_PER_W
    bufs = (
        (src0, dst0, hi0, lo0, rows0, oh0, semx0, seme0, semss0, semsc0),
        (src1, dst1, hi1, lo1, rows1, oh1, semx1, seme1, semss1, semsc1),
    )

    def _wait_scatters(buf):
        src_v, dst_v, hi_v, lo_v, rows_v, oh_v, semx, seme, semss, semsc = buf
        pltpu.make_async_copy(rows_v, sum_sh.at[dst_v], semss).wait()
        pltpu.make_async_copy(oh_v, cnt_sh.at[hi_v], semsc).wait()

    def _issue(i, buf, first):
        src_v, dst_v, hi_v, lo_v, rows_v, oh_v, semx, seme, semss, semsc = buf
        if not first:
            @pl.when(i >= 2)
            def _():
                _wait_scatters(buf)
        off = edge_base + i * CHUNK
        pltpu.sync_copy(src_hbm.at[pl.ds(off, CHUNK)], src_v)
        pltpu.sync_copy(dst_hbm.at[pl.ds(off, CHUNK)], dst_v)
        for j in range(CHUNK // _LANES):
            dst16 = dst_v[pl.ds(j * _LANES, _LANES)]
            hi_v[pl.ds(j * _LANES, _LANES)] = dst16 >> 7
            lo_v[pl.ds(j * _LANES, _LANES)] = dst16 & 127
        pltpu.async_copy(x_hbm.at[src_v], rows_v, semx)
        pltpu.async_copy(eye_sh.at[lo_v], oh_v, seme)

    def _drain(buf):
        src_v, dst_v, hi_v, lo_v, rows_v, oh_v, semx, seme, semss, semsc = buf
        pltpu.make_async_copy(x_hbm.at[src_v], rows_v, semx).wait()
        pltpu.async_copy(rows_v, sum_sh.at[dst_v], semss, add=True)
        pltpu.make_async_copy(eye_sh.at[lo_v], oh_v, seme).wait()
        pltpu.async_copy(oh_v, cnt_sh.at[hi_v], semsc, add=True)

    _issue(0, bufs[0], True)

    def _pair(g, carry):
        i0 = 2 * g
        _issue(i0 + 1, bufs[1], False)
        _drain(bufs[0])
        _issue(i0 + 2, bufs[0], False)
        _drain(bufs[1])
        return carry

    lax.fori_loop(0, (N_CHUNKS - 1) // 2, _pair, 0)
    _drain(bufs[0])
    _wait_scatters(bufs[1])
    _wait_scatters(bufs[0])
    plsc.subcore_barrier()

    pltpu.sync_copy(sum_sh.at[pl.ds(row_base, ROWS_PER_TILE)],
                    psum_hbm.at[c, pl.ds(row_base, ROWS_PER_TILE)])

    @pl.when(s < HROWS // 8)
    def _dump_cnt():
        pltpu.sync_copy(cnt_sh.at[pl.ds(s * 8, 8)],
                        pcnt_hbm.at[c, pl.ds(s * 8, 8)])


_sc_call = functools.partial(
    pl.kernel,
    mesh=plsc.VectorSubcoreMesh(core_axis_name="c", subcore_axis_name="s"),
    out_type=[
        jax.ShapeDtypeStruct((NC, N_PAD, D_FEAT), jnp.float32),
        jax.ShapeDtypeStruct((NC, HROWS, D_FEAT), jnp.float32),
    ],
    scratch_types=[
        pltpu.VMEM_SHARED((N_PAD, D_FEAT), jnp.float32),
        pltpu.VMEM_SHARED((HROWS, D_FEAT), jnp.float32),
        pltpu.VMEM_SHARED((D_FEAT, D_FEAT), jnp.float32),
        pltpu.VMEM((ZROWS, D_FEAT), jnp.float32),
    ] + 2 * [
        pltpu.VMEM((CHUNK,), jnp.int32),
        pltpu.VMEM((CHUNK,), jnp.int32),
        pltpu.VMEM((CHUNK,), jnp.int32),
        pltpu.VMEM((CHUNK,), jnp.int32),
        pltpu.VMEM((CHUNK, D_FEAT), jnp.float32),
        pltpu.VMEM((CHUNK, D_FEAT), jnp.float32),
        pltpu.SemaphoreType.DMA,
        pltpu.SemaphoreType.DMA,
        pltpu.SemaphoreType.DMA,
        pltpu.SemaphoreType.DMA,
    ],
)(_sc_body)


_ROWS_BLK = 400


def _divide_body(p0_ref, p1_ref, c0_ref, c1_ref, out_ref):
    cnt = c0_ref[...] + c1_ref[...]
    out_ref[...] = (p0_ref[...] + p1_ref[...]) / jnp.maximum(cnt, 1.0)


def kernel(X, edge_index):
    src = edge_index[0]
    dst = edge_index[1]
    eye = jnp.eye(D_FEAT, dtype=jnp.float32)
    psum, pcnt = _sc_call(X, src, dst, eye)
    c0 = pcnt[0].reshape(N_PAD, 1)
    c1 = pcnt[1].reshape(N_PAD, 1)
    out = pl.pallas_call(
        _divide_body,
        grid=(N_NODES // _ROWS_BLK,),
        in_specs=[
            pl.BlockSpec((_ROWS_BLK, D_FEAT), lambda i: (i, 0)),
            pl.BlockSpec((_ROWS_BLK, D_FEAT), lambda i: (i, 0)),
            pl.BlockSpec((_ROWS_BLK, 1), lambda i: (i, 0)),
            pl.BlockSpec((_ROWS_BLK, 1), lambda i: (i, 0)),
        ],
        out_specs=pl.BlockSpec((_ROWS_BLK, D_FEAT), lambda i: (i, 0)),
        out_shape=jax.ShapeDtypeStruct((N_NODES, D_FEAT), jnp.float32),
    )(psum[0], psum[1], c0, c1)
    return out
